# Initial kernel scaffold; baseline (speedup 1.0000x reference)
import jax
import jax.numpy as jnp
from jax.experimental import pallas as pl


def kernel(token_ids, table, token_prefix, ctx_embedding):
    raise NotImplementedError("write your pallas kernel here")


# speedup vs baseline: 1.0641x; 1.0641x over previous
import jax
import jax.numpy as jnp
from jax import lax
from jax.experimental import pallas as pl
from jax.experimental.pallas import tpu as pltpu
from jax.experimental.pallas import tpu_sc as plsc

_N_CLASSES = 1000
_D = 768
_CONTEXT_LENGTH = 77
_N_CTX = 17
_CTX_LEN = _N_CTX - 1
_SUFFIX_LEN = _CONTEXT_LENGTH - _N_CTX

_NW = 32
_CPW = 32
_BIG = 1 << 30


def _sc_body(token_ids_hbm, table_hbm, prefix_hbm, ctx_hbm,
             out_hbm, eos_hbm,
             idx_v, emb_v, eos_v, gsem):
    core = lax.axis_index("c")
    sub = lax.axis_index("s")
    wid = sub * 2 + core
    base_c = wid * _CPW

    pltpu.sync_copy(prefix_hbm, emb_v.at[pl.ds(0, 1)])
    pltpu.sync_copy(ctx_hbm, emb_v.at[pl.ds(1, _CTX_LEN)])

    lanes = lax.iota(jnp.int32, 16)
    lane0 = lanes == 0

    def body(g, carry):
        c = base_c + g

        @pl.when(c < _N_CLASSES)
        def _():
            pltpu.sync_copy(token_ids_hbm.at[c], idx_v)
            gather = pltpu.async_copy(
                table_hbm.at[idx_v], emb_v.at[pl.ds(_N_CTX, _SUFFIX_LEN)],
                gsem)

            gm = jnp.int32(-1)
            args = jnp.int32(_BIG)
            for off in (0, 16, 32, 44):
                ch = idx_v[pl.ds(off, 16)]
                gm = jnp.maximum(gm, jnp.max(ch))
            for off in (0, 16, 32, 44):
                ch = idx_v[pl.ds(off, 16)]
                gidx = lanes + off
                cand = jnp.where(ch == gm, gidx, _BIG)
                args = jnp.minimum(args, jnp.min(cand))
            eos = args + _N_CTX
            plsc.store_scatter(eos_v, [jnp.full((16,), g, jnp.int32)],
                               jnp.full((16,), eos, jnp.int32), mask=lane0)

            gather.wait()
            pltpu.sync_copy(emb_v, out_hbm.at[c])

        return carry

    lax.fori_loop(0, _CPW, body, 0)
    pltpu.sync_copy(eos_v, eos_hbm.at[pl.ds(base_c, _CPW)])


@jax.jit
def kernel(token_ids, table, token_prefix, ctx_embedding):
    mesh = plsc.VectorSubcoreMesh(core_axis_name="c", subcore_axis_name="s")
    run = pl.kernel(
        _sc_body,
        out_type=(
            jax.ShapeDtypeStruct((_N_CLASSES, _CONTEXT_LENGTH, _D),
                                 jnp.float32),
            jax.ShapeDtypeStruct((_NW * _CPW,), jnp.int32),
        ),
        mesh=mesh,
        compiler_params=pltpu.CompilerParams(
            use_tc_tiling_on_sc=False, needs_layout_passes=False),
        scratch_types=[
            pltpu.VMEM((_SUFFIX_LEN,), jnp.int32),
            pltpu.VMEM((_CONTEXT_LENGTH, _D), jnp.float32),
            pltpu.VMEM((_CPW,), jnp.int32),
            pltpu.SemaphoreType.DMA,
        ],
    )
    embeddings, eos_pad = run(token_ids, table, token_prefix, ctx_embedding)
    return embeddings, eos_pad[:_N_CLASSES]


# speedup vs baseline: 1.0841x; 1.0188x over previous
import jax
import jax.numpy as jnp
from jax import lax
from jax.experimental import pallas as pl
from jax.experimental.pallas import tpu as pltpu
from jax.experimental.pallas import tpu_sc as plsc

_N_CLASSES = 1000
_D = 768
_CONTEXT_LENGTH = 77
_N_CTX = 17
_CTX_LEN = _N_CTX - 1
_SUFFIX_LEN = _CONTEXT_LENGTH - _N_CTX

_NW = 32
_CPW = 32
_BIG = 1 << 30


def _sc_body(token_ids_hbm, table_hbm, prefix_hbm, ctx_hbm,
             out_hbm, eos_hbm,
             idx_v, emb0, emb1, eos_v, gsem0, gsem1, wsem0, wsem1):
    core = lax.axis_index("c")
    sub = lax.axis_index("s")
    wid = sub * 2 + core
    base_c = wid * _CPW

    bufs = (emb0, emb1)
    gsems = (gsem0, gsem1)
    wsems = (wsem0, wsem1)

    pltpu.sync_copy(token_ids_hbm.at[pl.ds(base_c, _CPW)], idx_v)
    for buf in bufs:
        pltpu.sync_copy(prefix_hbm, buf.at[pl.ds(0, 1)])
        pltpu.sync_copy(ctx_hbm, buf.at[pl.ds(1, _CTX_LEN)])

    lanes = lax.iota(jnp.int32, 16)
    lane0 = lanes == 0

    def gather_desc(g, b):
        return pltpu.make_async_copy(
            table_hbm.at[idx_v.at[g]],
            bufs[b].at[pl.ds(_N_CTX, _SUFFIX_LEN)],
            gsems[b])

    def write_desc(g, b):
        return pltpu.make_async_copy(bufs[b], out_hbm.at[base_c + g],
                                     wsems[b])

    gather_desc(0, 0).start()

    def half_iter(g, b):
        c = base_c + g

        @pl.when(jnp.logical_and(g + 1 < _CPW,
                                 base_c + g + 1 < _N_CLASSES))
        def _():
            @pl.when(g >= 1)
            def _():
                write_desc(g - 1, 1 - b).wait()
            gather_desc(g + 1, 1 - b).start()

        @pl.when(c < _N_CLASSES)
        def _():
            gm = jnp.int32(-1)
            args = jnp.int32(_BIG)
            for off in (0, 16, 32, 44):
                ch = idx_v[g, pl.ds(off, 16)]
                gm = jnp.maximum(gm, jnp.max(ch))
            for off in (0, 16, 32, 44):
                ch = idx_v[g, pl.ds(off, 16)]
                cand = jnp.where(ch == gm, lanes + off, _BIG)
                args = jnp.minimum(args, jnp.min(cand))
            plsc.store_scatter(eos_v, [jnp.full((16,), g, jnp.int32)],
                               jnp.full((16,), args + _N_CTX, jnp.int32),
                               mask=lane0)

            gather_desc(g, b).wait()
            write_desc(g, b).start()

    def body(i, carry):
        half_iter(2 * i, 0)
        half_iter(2 * i + 1, 1)
        return carry

    lax.fori_loop(0, _CPW // 2, body, 0)

    nvalid = jnp.minimum(_CPW, _N_CLASSES - base_c)
    write_desc(nvalid - 2, 0).wait()
    write_desc(nvalid - 1, 1).wait()

    pltpu.sync_copy(eos_v, eos_hbm.at[pl.ds(base_c, _CPW)])


@jax.jit
def kernel(token_ids, table, token_prefix, ctx_embedding):
    mesh = plsc.VectorSubcoreMesh(core_axis_name="c", subcore_axis_name="s")
    run = pl.kernel(
        _sc_body,
        out_type=(
            jax.ShapeDtypeStruct((_N_CLASSES, _CONTEXT_LENGTH, _D),
                                 jnp.float32),
            jax.ShapeDtypeStruct((_NW * _CPW,), jnp.int32),
        ),
        mesh=mesh,
        compiler_params=pltpu.CompilerParams(
            use_tc_tiling_on_sc=False, needs_layout_passes=False),
        scratch_types=[
            pltpu.VMEM((_CPW, _SUFFIX_LEN), jnp.int32),
            pltpu.VMEM((_CONTEXT_LENGTH, _D), jnp.float32),
            pltpu.VMEM((_CONTEXT_LENGTH, _D), jnp.float32),
            pltpu.VMEM((_CPW,), jnp.int32),
            pltpu.SemaphoreType.DMA,
            pltpu.SemaphoreType.DMA,
            pltpu.SemaphoreType.DMA,
            pltpu.SemaphoreType.DMA,
        ],
    )
    embeddings, eos_pad = run(token_ids, table, token_prefix, ctx_embedding)
    return embeddings, eos_pad[:_N_CLASSES]


# speedup vs baseline: 5.4818x; 5.0565x over previous
import jax
import jax.numpy as jnp
from jax import lax
from jax.experimental import pallas as pl
from jax.experimental.pallas import tpu as pltpu
from jax.experimental.pallas import tpu_sc as plsc

_N_CLASSES = 1000
_D = 768
_CONTEXT_LENGTH = 77
_N_CTX = 17
_CTX_LEN = _N_CTX - 1
_SUFFIX_LEN = _CONTEXT_LENGTH - _N_CTX

_NW = 32
_NCHUNK = _N_CLASSES // 8
_NJC = _D // 128
_BLK = 6 * 8
_BIG = 1 << 30


def _sc_body(ids_hbm, t2_hbm, pfx2_hbm, ctx2_hbm,
             out2_hbm, eos_hbm,
             headk, gbuf0, gbuf1, idsv, idxcb, ib0, ib1, eosv,
             hgsem, hwsem, gsem0, gsem1, wsem0, wsem1):
    core = lax.axis_index("c")
    sub = lax.axis_index("s")
    wid = sub * 2 + core
    nch = jnp.where(wid <= 28, 4, 3)

    gbufs = (gbuf0, gbuf1)
    ibs = (ib0, ib1)
    gsems = (gsem0, gsem1)
    wsems = (wsem0, wsem1)

    lanes = lax.iota(jnp.int32, 16)
    lane0 = lanes == 0
    lane_s = lanes & 7
    lane_jc8 = (lanes >> 3) * 8

    for ci in range(4):
        @pl.when(ci < nch)
        def _():
            pltpu.sync_copy(ids_hbm.at[pl.ds((wid + 32 * ci) * 8, 8)],
                            idsv.at[pl.ds(ci * 8, 8)])

    for t in range(_N_CTX):
        q = t - 1
        for k in range(3):
            if t == 0:
                vec = 2 * k + (lanes >> 3)
            else:
                vec = (q >> 3) * _BLK + 16 * k + lane_jc8 + (q & 7)
            idxcb[pl.ds(_BLK * t + 16 * k, 16)] = vec
    for t in range(_N_CTX):
        src = pfx2_hbm if t == 0 else ctx2_hbm
        pltpu.async_copy(src.at[idxcb.at[pl.ds(_BLK * t, _BLK)]],
                         headk.at[pl.ds(_BLK * t, _BLK)], hgsem)
    for t in range(_N_CTX):
        src = pfx2_hbm if t == 0 else ctx2_hbm
        pltpu.make_async_copy(src.at[idxcb.at[pl.ds(_BLK * t, _BLK)]],
                              headk.at[pl.ds(_BLK * t, _BLK)], hgsem).wait()

    def head_write_desc(t, ci):
        crg = wid + 32 * ci
        return pltpu.make_async_copy(
            headk.at[pl.ds(_BLK * t, _BLK)],
            out2_hbm.at[pl.ds(_BLK * (t * _NCHUNK + crg), _BLK)], hwsem)

    for t in range(_N_CTX):
        for ci in range(4):
            @pl.when(ci < nch)
            def _():
                head_write_desc(t, ci).start()

    def eos_body(cls, carry):
        gm = jnp.int32(-1)
        args = jnp.int32(_BIG)
        for off in (0, 16, 32, 44):
            ch = idsv[cls, pl.ds(off, 16)]
            gm = jnp.maximum(gm, jnp.max(ch))
        for off in (0, 16, 32, 44):
            ch = idsv[cls, pl.ds(off, 16)]
            cand = jnp.where(ch == gm, lanes + off, _BIG)
            args = jnp.minimum(args, jnp.min(cand))
        plsc.store_scatter(eosv, [jnp.full((16,), cls, jnp.int32)],
                           jnp.full((16,), args + _N_CTX, jnp.int32),
                           mask=lane0)
        return carry

    lax.fori_loop(0, nch * 8, eos_body, 0)
    for ci in range(4):
        @pl.when(ci < nch)
        def _():
            pltpu.sync_copy(eosv.at[pl.ds(ci * 8, 8)],
                            eos_hbm.at[pl.ds((wid + 32 * ci) * 8, 8)])

    nu = _SUFFIX_LEN * nch

    def build_idx(u, b):
        j = u // nch
        ci = u % nch
        rv = plsc.load_gather(idsv, [ci * 8 + lane_s,
                                     jnp.full((16,), j, jnp.int32)])
        hi = (rv >> 3) * _BLK + (rv & 7)
        for k in range(3):
            ibs[b][pl.ds(16 * k, 16)] = hi + 16 * k + lane_jc8

    def gather_desc(b):
        return pltpu.make_async_copy(t2_hbm.at[ibs[b]], gbufs[b], gsems[b])

    def write_desc(u, b):
        j = u // nch
        ci = u % nch
        crg = wid + 32 * ci
        return pltpu.make_async_copy(
            gbufs[b],
            out2_hbm.at[pl.ds(_BLK * ((_N_CTX + j) * _NCHUNK + crg), _BLK)],
            wsems[b])

    build_idx(0, 0)
    gather_desc(0).start()

    def half_iter(u, b):
        @pl.when(u + 1 < nu)
        def _():
            @pl.when(u >= 1)
            def _():
                write_desc(u - 1, 1 - b).wait()
            build_idx(u + 1, 1 - b)
            gather_desc(1 - b).start()

        gather_desc(b).wait()
        write_desc(u, b).start()

    def body(i, carry):
        half_iter(2 * i, 0)
        half_iter(2 * i + 1, 1)
        return carry

    lax.fori_loop(0, nu // 2, body, 0)

    write_desc(nu - 2, 0).wait()
    write_desc(nu - 1, 1).wait()

    for t in range(_N_CTX):
        for ci in range(4):
            @pl.when(ci < nch)
            def _():
                head_write_desc(t, ci).wait()


@jax.jit
def kernel(token_ids, table, token_prefix, ctx_embedding):
    t2 = (table.reshape(49408 // 8, 8, _NJC, 128)
          .transpose(0, 2, 1, 3).reshape(49408 // 8 * _NJC * 8, 128))
    pfx2 = token_prefix.reshape(_NJC, 128)
    ctx2 = (ctx_embedding.reshape(2, 8, _NJC, 128)
            .transpose(0, 2, 1, 3).reshape(2 * _NJC * 8, 128))

    mesh = plsc.VectorSubcoreMesh(core_axis_name="c", subcore_axis_name="s")
    run = pl.kernel(
        _sc_body,
        out_type=(
            jax.ShapeDtypeStruct((_CONTEXT_LENGTH * _NCHUNK * _BLK, 128),
                                 jnp.float32),
            jax.ShapeDtypeStruct((_N_CLASSES,), jnp.int32),
        ),
        mesh=mesh,
        compiler_params=pltpu.CompilerParams(
            use_tc_tiling_on_sc=False, needs_layout_passes=False),
        scratch_types=[
            pltpu.VMEM((_N_CTX * _BLK, 128), jnp.float32),
            pltpu.VMEM((_BLK, 128), jnp.float32),
            pltpu.VMEM((_BLK, 128), jnp.float32),
            pltpu.VMEM((32, _SUFFIX_LEN), jnp.int32),
            pltpu.VMEM((_N_CTX * _BLK,), jnp.int32),
            pltpu.VMEM((_BLK,), jnp.int32),
            pltpu.VMEM((_BLK,), jnp.int32),
            pltpu.VMEM((32,), jnp.int32),
            pltpu.SemaphoreType.DMA,
            pltpu.SemaphoreType.DMA,
            pltpu.SemaphoreType.DMA,
            pltpu.SemaphoreType.DMA,
            pltpu.SemaphoreType.DMA,
            pltpu.SemaphoreType.DMA,
        ],
    )
    out2, eos = run(token_ids, t2, pfx2, ctx2)
    embeddings = (out2.reshape(_CONTEXT_LENGTH, _NCHUNK, _NJC, 8, 128)
                  .transpose(1, 3, 0, 2, 4)
                  .reshape(_N_CLASSES, _CONTEXT_LENGTH, _D))
    return embeddings, eos


# speedup vs baseline: 8.4458x; 1.5407x over previous
import jax
import jax.numpy as jnp
from jax import lax
from jax.experimental import pallas as pl
from jax.experimental.pallas import tpu as pltpu
from jax.experimental.pallas import tpu_sc as plsc

_N_CLASSES = 1000
_D = 768
_CONTEXT_LENGTH = 77
_N_CTX = 17
_CTX_LEN = _N_CTX - 1
_SUFFIX_LEN = _CONTEXT_LENGTH - _N_CTX

_NW = 32
_NCHUNK = _N_CLASSES // 8
_NJC = _D // 128
_BLK = 6 * 8
_NBUF = 6
_PREP = 3
_BIG = 1 << 30


def _sc_body(ids_hbm, t2_hbm, pfx2_hbm, ctx2_hbm,
             out2_hbm, eos_hbm,
             headk, headtmp, gb0, gb1, gb2, gb3, gb4, gb5,
             idsv, idxcb, ib0, ib1, ib2, ib3, ib4, ib5, eosv,
             hgsem, hwsem, gsemA, gsemB, gsemC, gsemD, gsemE, gsemF,
             wsemA, wsemB, wsemC, wsemD, wsemE, wsemF):
    core = lax.axis_index("c")
    sub = lax.axis_index("s")
    wid = sub * 2 + core
    nch = jnp.where(wid <= 28, 4, 3)

    gbufs = (gb0, gb1, gb2, gb3, gb4, gb5)
    ibs = (ib0, ib1, ib2, ib3, ib4, ib5)
    gsems = (gsemA, gsemB, gsemC, gsemD, gsemE, gsemF)
    wsems = (wsemA, wsemB, wsemC, wsemD, wsemE, wsemF)

    lanes = lax.iota(jnp.int32, 16)
    lane0 = lanes == 0
    lane_s = lanes & 7
    lane_jc8 = (lanes >> 3) * 8

    for ci in range(4):
        @pl.when(ci < nch)
        def _():
            pltpu.sync_copy(ids_hbm.at[pl.ds((wid + 32 * ci) * 8, 8)],
                            idsv.at[pl.ds(ci * 8, 8)])

    for t in range(_N_CTX):
        q = t - 1
        for k in range(3):
            if t == 0:
                vec = 2 * k + (lanes >> 3)
            else:
                vec = (q >> 3) * _BLK + 16 * k + lane_jc8 + (q & 7)
            idxcb[pl.ds(_BLK * t + 16 * k, 16)] = vec

    def stage_head(t):
        src = pfx2_hbm if t == 0 else ctx2_hbm
        pltpu.async_copy(src.at[idxcb.at[pl.ds(_BLK * t, _BLK)]],
                         headtmp, hgsem).wait()
        pltpu.sync_copy(headtmp, headk.at[pl.ds(_BLK * t, _BLK)])

    for t in range(16):
        @pl.when(sub == t)
        def _():
            stage_head(t)

    @pl.when(sub == 0)
    def _():
        stage_head(16)

    plsc.subcore_barrier()

    def head_write_desc(t, ci):
        crg = wid + 32 * ci
        return pltpu.make_async_copy(
            headk.at[pl.ds(_BLK * t, _BLK)],
            out2_hbm.at[pl.ds(_BLK * (t * _NCHUNK + crg), _BLK)], hwsem)

    for t in range(_N_CTX):
        for ci in range(4):
            @pl.when(ci < nch)
            def _():
                head_write_desc(t, ci).start()

    def eos_body(cls, carry):
        gm = jnp.int32(-1)
        args = jnp.int32(_BIG)
        for off in (0, 16, 32, 44):
            ch = idsv[cls, pl.ds(off, 16)]
            gm = jnp.maximum(gm, jnp.max(ch))
        for off in (0, 16, 32, 44):
            ch = idsv[cls, pl.ds(off, 16)]
            cand = jnp.where(ch == gm, lanes + off, _BIG)
            args = jnp.minimum(args, jnp.min(cand))
        plsc.store_scatter(eosv, [jnp.full((16,), cls, jnp.int32)],
                           jnp.full((16,), args + _N_CTX, jnp.int32),
                           mask=lane0)
        return carry

    lax.fori_loop(0, nch * 8, eos_body, 0)
    for ci in range(4):
        @pl.when(ci < nch)
        def _():
            pltpu.sync_copy(eosv.at[pl.ds(ci * 8, 8)],
                            eos_hbm.at[pl.ds((wid + 32 * ci) * 8, 8)])

    nu = _SUFFIX_LEN * nch

    def build_idx(u, b):
        j = u // nch
        ci = u % nch
        rv = plsc.load_gather(idsv, [ci * 8 + lane_s,
                                     jnp.full((16,), j, jnp.int32)])
        hi = (rv >> 3) * _BLK + (rv & 7)
        for k in range(3):
            ibs[b][pl.ds(16 * k, 16)] = hi + 16 * k + lane_jc8

    def gather_desc(b):
        return pltpu.make_async_copy(t2_hbm.at[ibs[b]], gbufs[b], gsems[b])

    def write_desc(u, b):
        j = u // nch
        ci = u % nch
        crg = wid + 32 * ci
        return pltpu.make_async_copy(
            gbufs[b],
            out2_hbm.at[pl.ds(_BLK * ((_N_CTX + j) * _NCHUNK + crg), _BLK)],
            wsems[b])

    for b in range(_PREP):
        build_idx(b, b)
        gather_desc(b).start()

    def ring_iter(u, b):
        @pl.when(u + _PREP < nu)
        def _():
            @pl.when(u >= _NBUF - _PREP)
            def _():
                write_desc(u + _PREP - _NBUF, (b + _PREP) % _NBUF).wait()
            build_idx(u + _PREP, (b + _PREP) % _NBUF)
            gather_desc((b + _PREP) % _NBUF).start()

        gather_desc(b).wait()
        write_desc(u, b).start()

    def body(i, carry):
        for b in range(_NBUF):
            ring_iter(_NBUF * i + b, b)
        return carry

    lax.fori_loop(0, nu // _NBUF, body, 0)

    for b in range(_NBUF):
        write_desc(nu - _NBUF + b, b).wait()

    for t in range(_N_CTX):
        for ci in range(4):
            @pl.when(ci < nch)
            def _():
                head_write_desc(t, ci).wait()


@jax.jit
def kernel(token_ids, table, token_prefix, ctx_embedding):
    t2 = (table.reshape(49408 // 8, 8, _NJC, 128)
          .transpose(0, 2, 1, 3).reshape(49408 // 8 * _NJC * 8, 128))
    pfx2 = token_prefix.reshape(_NJC, 128)
    ctx2 = (ctx_embedding.reshape(2, 8, _NJC, 128)
            .transpose(0, 2, 1, 3).reshape(2 * _NJC * 8, 128))

    mesh = plsc.VectorSubcoreMesh(core_axis_name="c", subcore_axis_name="s")
    run = pl.kernel(
        _sc_body,
        out_type=(
            jax.ShapeDtypeStruct((_CONTEXT_LENGTH * _NCHUNK * _BLK, 128),
                                 jnp.float32),
            jax.ShapeDtypeStruct((_N_CLASSES,), jnp.int32),
        ),
        mesh=mesh,
        compiler_params=pltpu.CompilerParams(
            use_tc_tiling_on_sc=False, needs_layout_passes=False),
        scratch_types=[
            pltpu.VMEM_SHARED((_N_CTX * _BLK, 128), jnp.float32),
            pltpu.VMEM((_BLK, 128), jnp.float32),
            pltpu.VMEM((_BLK, 128), jnp.float32),
            pltpu.VMEM((_BLK, 128), jnp.float32),
            pltpu.VMEM((_BLK, 128), jnp.float32),
            pltpu.VMEM((_BLK, 128), jnp.float32),
            pltpu.VMEM((_BLK, 128), jnp.float32),
            pltpu.VMEM((_BLK, 128), jnp.float32),
            pltpu.VMEM((32, _SUFFIX_LEN), jnp.int32),
            pltpu.VMEM((_N_CTX * _BLK,), jnp.int32),
            pltpu.VMEM((_BLK,), jnp.int32),
            pltpu.VMEM((_BLK,), jnp.int32),
            pltpu.VMEM((_BLK,), jnp.int32),
            pltpu.VMEM((_BLK,), jnp.int32),
            pltpu.VMEM((_BLK,), jnp.int32),
            pltpu.VMEM((_BLK,), jnp.int32),
            pltpu.VMEM((32,), jnp.int32),
            pltpu.SemaphoreType.DMA,
            pltpu.SemaphoreType.DMA,
            pltpu.SemaphoreType.DMA,
            pltpu.SemaphoreType.DMA,
            pltpu.SemaphoreType.DMA,
            pltpu.SemaphoreType.DMA,
            pltpu.SemaphoreType.DMA,
            pltpu.SemaphoreType.DMA,
            pltpu.SemaphoreType.DMA,
            pltpu.SemaphoreType.DMA,
            pltpu.SemaphoreType.DMA,
            pltpu.SemaphoreType.DMA,
            pltpu.SemaphoreType.DMA,
            pltpu.SemaphoreType.DMA,
        ],
    )
    out2, eos = run(token_ids, t2, pfx2, ctx2)
    embeddings = (out2.reshape(_CONTEXT_LENGTH, _NCHUNK, _NJC, 8, 128)
                  .transpose(1, 3, 0, 2, 4)
                  .reshape(_N_CLASSES, _CONTEXT_LENGTH, _D))
    return embeddings, eos


# speedup vs baseline: 8.4909x; 1.0053x over previous
import jax
import jax.numpy as jnp
from jax import lax
from jax.experimental import pallas as pl
from jax.experimental.pallas import tpu as pltpu
from jax.experimental.pallas import tpu_sc as plsc

_N_CLASSES = 1000
_D = 768
_CONTEXT_LENGTH = 77
_N_CTX = 17
_CTX_LEN = _N_CTX - 1
_SUFFIX_LEN = _CONTEXT_LENGTH - _N_CTX

_NW = 32
_NCHUNK = _N_CLASSES // 8
_NJC = _D // 128
_BLK = 6 * 8
_NBUF = 10
_PREP = 5
_BIG = 1 << 30


def _sc_body(ids_hbm, t2_hbm, pfx2_hbm, ctx2_hbm,
             out2_hbm, eos_hbm,
             headk, headtmp, gbufs_t, ibs_t, idsv, idxcb, eosv,
             hgsem, hwsem, gsems_t, wsems_t):
    core = lax.axis_index("c")
    sub = lax.axis_index("s")
    wid = sub * 2 + core
    nch = jnp.where(wid <= 28, 4, 3)

    gbufs = tuple(gbufs_t)
    ibs = tuple(ibs_t)
    gsems = tuple(gsems_t)
    wsems = tuple(wsems_t)

    lanes = lax.iota(jnp.int32, 16)
    lane0 = lanes == 0
    lane_s = lanes & 7
    lane_jc8 = (lanes >> 3) * 8

    for ci in range(4):
        @pl.when(ci < nch)
        def _():
            pltpu.sync_copy(ids_hbm.at[pl.ds((wid + 32 * ci) * 8, 8)],
                            idsv.at[pl.ds(ci * 8, 8)])

    for t in range(_N_CTX):
        q = t - 1
        for k in range(3):
            if t == 0:
                vec = 2 * k + (lanes >> 3)
            else:
                vec = (q >> 3) * _BLK + 16 * k + lane_jc8 + (q & 7)
            idxcb[pl.ds(_BLK * t + 16 * k, 16)] = vec

    def stage_head(t):
        src = pfx2_hbm if t == 0 else ctx2_hbm
        pltpu.async_copy(src.at[idxcb.at[pl.ds(_BLK * t, _BLK)]],
                         headtmp, hgsem).wait()
        pltpu.sync_copy(headtmp, headk.at[pl.ds(_BLK * t, _BLK)])

    for t in range(16):
        @pl.when(sub == t)
        def _():
            stage_head(t)

    @pl.when(sub == 0)
    def _():
        stage_head(16)

    plsc.subcore_barrier()

    def head_write_desc(t, ci):
        crg = wid + 32 * ci
        return pltpu.make_async_copy(
            headk.at[pl.ds(_BLK * t, _BLK)],
            out2_hbm.at[pl.ds(_BLK * (t * _NCHUNK + crg), _BLK)], hwsem)

    for t in range(_N_CTX):
        for ci in range(4):
            @pl.when(ci < nch)
            def _():
                head_write_desc(t, ci).start()

    def eos_body(cls, carry):
        gm = jnp.int32(-1)
        args = jnp.int32(_BIG)
        for off in (0, 16, 32, 44):
            ch = idsv[cls, pl.ds(off, 16)]
            gm = jnp.maximum(gm, jnp.max(ch))
        for off in (0, 16, 32, 44):
            ch = idsv[cls, pl.ds(off, 16)]
            cand = jnp.where(ch == gm, lanes + off, _BIG)
            args = jnp.minimum(args, jnp.min(cand))
        plsc.store_scatter(eosv, [jnp.full((16,), cls, jnp.int32)],
                           jnp.full((16,), args + _N_CTX, jnp.int32),
                           mask=lane0)
        return carry

    lax.fori_loop(0, nch * 8, eos_body, 0)
    for ci in range(4):
        @pl.when(ci < nch)
        def _():
            pltpu.sync_copy(eosv.at[pl.ds(ci * 8, 8)],
                            eos_hbm.at[pl.ds((wid + 32 * ci) * 8, 8)])

    nu = _SUFFIX_LEN * nch

    def build_idx(u, b):
        j = u // nch
        ci = u % nch
        rv = plsc.load_gather(idsv, [ci * 8 + lane_s,
                                     jnp.full((16,), j, jnp.int32)])
        hi = (rv >> 3) * _BLK + (rv & 7)
        for k in range(3):
            ibs[b][pl.ds(16 * k, 16)] = hi + 16 * k + lane_jc8

    def gather_desc(b):
        return pltpu.make_async_copy(t2_hbm.at[ibs[b]], gbufs[b], gsems[b])

    def write_desc(u, b):
        j = u // nch
        ci = u % nch
        crg = wid + 32 * ci
        return pltpu.make_async_copy(
            gbufs[b],
            out2_hbm.at[pl.ds(_BLK * ((_N_CTX + j) * _NCHUNK + crg), _BLK)],
            wsems[b])

    for b in range(_PREP):
        build_idx(b, b)
        gather_desc(b).start()

    def ring_iter(u, b):
        @pl.when(u + _PREP < nu)
        def _():
            @pl.when(u >= _NBUF - _PREP)
            def _():
                write_desc(u + _PREP - _NBUF, (b + _PREP) % _NBUF).wait()
            build_idx(u + _PREP, (b + _PREP) % _NBUF)
            gather_desc((b + _PREP) % _NBUF).start()

        gather_desc(b).wait()
        write_desc(u, b).start()

    def body(i, carry):
        for b in range(_NBUF):
            ring_iter(_NBUF * i + b, b)
        return carry

    lax.fori_loop(0, nu // _NBUF, body, 0)

    for b in range(_NBUF):
        write_desc(nu - _NBUF + b, b).wait()

    for t in range(_N_CTX):
        for ci in range(4):
            @pl.when(ci < nch)
            def _():
                head_write_desc(t, ci).wait()


@jax.jit
def kernel(token_ids, table, token_prefix, ctx_embedding):
    t2 = (table.reshape(49408 // 8, 8, _NJC, 128)
          .transpose(0, 2, 1, 3).reshape(49408 // 8 * _NJC * 8, 128))
    pfx2 = token_prefix.reshape(_NJC, 128)
    ctx2 = (ctx_embedding.reshape(2, 8, _NJC, 128)
            .transpose(0, 2, 1, 3).reshape(2 * _NJC * 8, 128))

    mesh = plsc.VectorSubcoreMesh(core_axis_name="c", subcore_axis_name="s")
    run = pl.kernel(
        _sc_body,
        out_type=(
            jax.ShapeDtypeStruct((_CONTEXT_LENGTH * _NCHUNK * _BLK, 128),
                                 jnp.float32),
            jax.ShapeDtypeStruct((_N_CLASSES,), jnp.int32),
        ),
        mesh=mesh,
        compiler_params=pltpu.CompilerParams(
            use_tc_tiling_on_sc=False, needs_layout_passes=False),
        scratch_types=[
            pltpu.VMEM_SHARED((_N_CTX * _BLK, 128), jnp.float32),
            pltpu.VMEM((_BLK, 128), jnp.float32),
            [pltpu.VMEM((_BLK, 128), jnp.float32)] * _NBUF,
            [pltpu.VMEM((_BLK,), jnp.int32)] * _NBUF,
            pltpu.VMEM((32, _SUFFIX_LEN), jnp.int32),
            pltpu.VMEM((_N_CTX * _BLK,), jnp.int32),
            pltpu.VMEM((32,), jnp.int32),
            pltpu.SemaphoreType.DMA,
            pltpu.SemaphoreType.DMA,
            [pltpu.SemaphoreType.DMA] * _NBUF,
            [pltpu.SemaphoreType.DMA] * _NBUF,
        ],
    )
    out2, eos = run(token_ids, t2, pfx2, ctx2)
    embeddings = (out2.reshape(_CONTEXT_LENGTH, _NCHUNK, _NJC, 8, 128)
                  .transpose(1, 3, 0, 2, 4)
                  .reshape(_N_CLASSES, _CONTEXT_LENGTH, _D))
    return embeddings, eos


# speedup vs baseline: 8.5544x; 1.0075x over previous
import jax
import jax.numpy as jnp
from jax import lax
from jax.experimental import pallas as pl
from jax.experimental.pallas import tpu as pltpu
from jax.experimental.pallas import tpu_sc as plsc

_N_CLASSES = 1000
_D = 768
_CONTEXT_LENGTH = 77
_N_CTX = 17
_CTX_LEN = _N_CTX - 1
_SUFFIX_LEN = _CONTEXT_LENGTH - _N_CTX

_NCHUNK = _N_CLASSES // 8
_NJC = _D // 128
_BLK = 6 * 8
_PBLK = 2 * _BLK
_NBUF = 6
_PREP = 3
_BIG = 1 << 30


def _sc_body(ids_hbm, t2_hbm, pfx2_hbm, ctx2_hbm,
             out2_hbm, eos_hbm,
             headk, headtmp, gbufs_t, ibs_t, idsv, idxcb, eosv,
             hgsem, hwsem, gsems_t, wsems_t):
    core = lax.axis_index("c")
    sub = lax.axis_index("s")
    wid = sub * 2 + core
    nseg = jnp.where(wid < 30, 2, 1)

    gbufs = tuple(gbufs_t)
    ibs = tuple(ibs_t)
    gsems = tuple(gsems_t)
    wsems = tuple(wsems_t)

    lanes = lax.iota(jnp.int32, 16)
    lane0 = lanes == 0
    lane_s = lanes & 7
    lane_jc8 = (lanes >> 3) * 8

    def pair_of(seg):
        return jnp.where((seg == 0) & (wid < 30), wid, wid + 30)

    for seg in range(2):
        @pl.when(seg < nseg)
        def _():
            pltpu.sync_copy(ids_hbm.at[pl.ds(16 * pair_of(seg), 16)],
                            idsv.at[pl.ds(16 * seg, 16)])

    @pl.when(wid >= 30)
    def _():
        pltpu.sync_copy(ids_hbm.at[pl.ds(992, 8)], idsv.at[pl.ds(16, 8)])

    for t in range(_N_CTX):
        q = t - 1
        for k in range(3):
            if t == 0:
                vec = 2 * k + (lanes >> 3)
            else:
                vec = (q >> 3) * _BLK + 16 * k + lane_jc8 + (q & 7)
            idxcb[pl.ds(_BLK * t + 16 * k, 16)] = vec

    def stage_head(t):
        src = pfx2_hbm if t == 0 else ctx2_hbm
        pltpu.async_copy(src.at[idxcb.at[pl.ds(_BLK * t, _BLK)]],
                         headtmp, hgsem).wait()
        pltpu.sync_copy(headtmp, headk.at[pl.ds(_PBLK * t, _BLK)])
        pltpu.sync_copy(headtmp, headk.at[pl.ds(_PBLK * t + _BLK, _BLK)])

    for t in range(16):
        @pl.when(sub == t)
        def _():
            stage_head(t)

    @pl.when(sub == 0)
    def _():
        stage_head(16)

    plsc.subcore_barrier()

    def head_write_desc(t, seg):
        return pltpu.make_async_copy(
            headk.at[pl.ds(_PBLK * t, _PBLK)],
            out2_hbm.at[pl.ds(_BLK * (t * _NCHUNK + 2 * pair_of(seg)),
                              _PBLK)],
            hwsem)

    def head_write124_desc(t):
        return pltpu.make_async_copy(
            headk.at[pl.ds(_PBLK * t, _BLK)],
            out2_hbm.at[pl.ds(_BLK * (t * _NCHUNK + 124), _BLK)], hwsem)

    for t in range(_N_CTX):
        for seg in range(2):
            @pl.when(seg < nseg)
            def _():
                head_write_desc(t, seg).start()

    @pl.when(wid == 31)
    def _():
        for t in range(_N_CTX):
            head_write124_desc(t).start()

    def eos_body(cls, carry):
        gm = jnp.int32(-1)
        args = jnp.int32(_BIG)
        for off in (0, 16, 32, 44):
            ch = idsv[cls, pl.ds(off, 16)]
            gm = jnp.maximum(gm, jnp.max(ch))
        for off in (0, 16, 32, 44):
            ch = idsv[cls, pl.ds(off, 16)]
            cand = jnp.where(ch == gm, lanes + off, _BIG)
            args = jnp.minimum(args, jnp.min(cand))
        plsc.store_scatter(eosv, [jnp.full((16,), cls, jnp.int32)],
                           jnp.full((16,), args + _N_CTX, jnp.int32),
                           mask=lane0)
        return carry

    lax.fori_loop(0, nseg * 16, eos_body, 0)
    for seg in range(2):
        @pl.when(seg < nseg)
        def _():
            pltpu.sync_copy(eosv.at[pl.ds(16 * seg, 16)],
                            eos_hbm.at[pl.ds(16 * pair_of(seg), 16)])

    @pl.when(wid == 30)
    def _():
        lax.fori_loop(16, 24, eos_body, 0)
        pltpu.sync_copy(eosv.at[pl.ds(16, 8)], eos_hbm.at[pl.ds(992, 8)])

    nu = _SUFFIX_LEN * nseg

    def build_idx(u, b):
        j = u // nseg
        seg = u % nseg
        jv = jnp.full((16,), j, jnp.int32)
        for half in range(2):
            rv = plsc.load_gather(idsv, [16 * seg + 8 * half + lane_s, jv])
            hi = (rv >> 3) * _BLK + (rv & 7)
            for kk in range(3):
                ibs[b][pl.ds(_BLK * half + 16 * kk, 16)] = (
                    hi + 16 * kk + lane_jc8)

    def gather_desc(b):
        return pltpu.make_async_copy(t2_hbm.at[ibs[b]], gbufs[b], gsems[b])

    def write_desc(u, b):
        j = u // nseg
        seg = u % nseg
        return pltpu.make_async_copy(
            gbufs[b],
            out2_hbm.at[pl.ds(
                _BLK * ((_N_CTX + j) * _NCHUNK + 2 * pair_of(seg)), _PBLK)],
            wsems[b])

    for b in range(_PREP):
        build_idx(b, b)
        gather_desc(b).start()

    def ring_iter(u, b):
        @pl.when(u + _PREP < nu)
        def _():
            @pl.when(u >= _NBUF - _PREP)
            def _():
                write_desc(u + _PREP - _NBUF, (b + _PREP) % _NBUF).wait()
            build_idx(u + _PREP, (b + _PREP) % _NBUF)
            gather_desc((b + _PREP) % _NBUF).start()

        gather_desc(b).wait()
        write_desc(u, b).start()

    def body(i, carry):
        for b in range(_NBUF):
            ring_iter(_NBUF * i + b, b)
        return carry

    lax.fori_loop(0, nu // _NBUF, body, 0)

    for b in range(_NBUF):
        write_desc(nu - _NBUF + b, b).wait()

    @pl.when(wid >= 30)
    def _():
        mbase = 15 * (wid - 30)

        def e_build(m, b):
            for half in range(2):
                jv = jnp.full((16,), 2 * (mbase + m) + half, jnp.int32)
                rv = plsc.load_gather(idsv, [16 + lane_s, jv])
                hi = (rv >> 3) * _BLK + (rv & 7)
                for kk in range(3):
                    ibs[b][pl.ds(_BLK * half + 16 * kk, 16)] = (
                        hi + 16 * kk + lane_jc8)

        def e_gather(b):
            return pltpu.make_async_copy(t2_hbm.at[ibs[b]], gbufs[b],
                                         gsems[b])

        def e_write(m, b, half):
            t = _N_CTX + 2 * (mbase + m) + half
            return pltpu.make_async_copy(
                gbufs[b].at[pl.ds(_BLK * half, _BLK)],
                out2_hbm.at[pl.ds(_BLK * (t * _NCHUNK + 124), _BLK)],
                wsems[b])

        e_build(0, 0)
        e_gather(0).start()
        for m in range(15):
            b = m % 2
            if m + 1 < 15:
                if m >= 1:
                    e_write(m - 1, 1 - b, 0).wait()
                    e_write(m - 1, 1 - b, 1).wait()
                e_build(m + 1, 1 - b)
                e_gather(1 - b).start()
            e_gather(b).wait()
            e_write(m, b, 0).start()
            e_write(m, b, 1).start()
        e_write(13, 1, 0).wait()
        e_write(13, 1, 1).wait()
        e_write(14, 0, 0).wait()
        e_write(14, 0, 1).wait()

    for t in range(_N_CTX):
        for seg in range(2):
            @pl.when(seg < nseg)
            def _():
                head_write_desc(t, seg).wait()

    @pl.when(wid == 31)
    def _():
        for t in range(_N_CTX):
            head_write124_desc(t).wait()


@jax.jit
def kernel(token_ids, table, token_prefix, ctx_embedding):
    t2 = (table.reshape(49408 // 8, 8, _NJC, 128)
          .transpose(0, 2, 1, 3).reshape(49408 // 8 * _NJC * 8, 128))
    pfx2 = token_prefix.reshape(_NJC, 128)
    ctx2 = (ctx_embedding.reshape(2, 8, _NJC, 128)
            .transpose(0, 2, 1, 3).reshape(2 * _NJC * 8, 128))

    mesh = plsc.VectorSubcoreMesh(core_axis_name="c", subcore_axis_name="s")
    run = pl.kernel(
        _sc_body,
        out_type=(
            jax.ShapeDtypeStruct((_CONTEXT_LENGTH * _NCHUNK * _BLK, 128),
                                 jnp.float32),
            jax.ShapeDtypeStruct((_N_CLASSES,), jnp.int32),
        ),
        mesh=mesh,
        compiler_params=pltpu.CompilerParams(
            use_tc_tiling_on_sc=False, needs_layout_passes=False),
        scratch_types=[
            pltpu.VMEM_SHARED((_N_CTX * _PBLK, 128), jnp.float32),
            pltpu.VMEM((_BLK, 128), jnp.float32),
            [pltpu.VMEM((_PBLK, 128), jnp.float32)] * _NBUF,
            [pltpu.VMEM((_PBLK,), jnp.int32)] * _NBUF,
            pltpu.VMEM((32, _SUFFIX_LEN), jnp.int32),
            pltpu.VMEM((_N_CTX * _BLK,), jnp.int32),
            pltpu.VMEM((32,), jnp.int32),
            pltpu.SemaphoreType.DMA,
            pltpu.SemaphoreType.DMA,
            [pltpu.SemaphoreType.DMA] * _NBUF,
            [pltpu.SemaphoreType.DMA] * _NBUF,
        ],
    )
    out2, eos = run(token_ids, t2, pfx2, ctx2)
    embeddings = (out2.reshape(_CONTEXT_LENGTH, _NCHUNK, _NJC, 8, 128)
                  .transpose(1, 3, 0, 2, 4)
                  .reshape(_N_CLASSES, _CONTEXT_LENGTH, _D))
    return embeddings, eos
